# baseline (speedup 1.0000x reference)
import functools

import jax
import jax.numpy as jnp
from jax import lax
from jax.experimental import pallas as pl
from jax.experimental.pallas import tpu as pltpu
from jax.experimental.pallas import tpu_sc as plsc

N = 10000
D = 128
DH = 64
HOPS = 4
E = 320000

NSUB = 16
NPAD = 10240
ROWS_PER_TILE = NPAD // NSUB
CHUNK = 128
CHUNKS_PER_TILE = 160
EPAD = NSUB * CHUNKS_PER_TILE * CHUNK
RSLICE = 64
NSLICES = ROWS_PER_TILE // RSLICE
GROUP = 16
NGROUPS = CHUNKS_PER_TILE // GROUP

_mesh = plsc.VectorSubcoreMesh(core_axis_name="c", subcore_axis_name="s")


@functools.partial(
    pl.kernel,
    out_type=jax.ShapeDtypeStruct((HOPS, 2, NPAD, DH), jnp.float32),
    mesh=_mesh,
    compiler_params=pltpu.CompilerParams(use_tc_tiling_on_sc=False),
    scratch_types=(
        pltpu.VMEM_SHARED((NPAD, DH), jnp.float32),
        pltpu.VMEM_SHARED((NPAD, DH), jnp.float32),
        pltpu.VMEM((ROWS_PER_TILE, 16), jnp.float32),
        pltpu.VMEM((HOPS, 16), jnp.float32),
        pltpu.VMEM((GROUP, CHUNK), jnp.int32),
        pltpu.VMEM((GROUP, CHUNK), jnp.int32),
        pltpu.VMEM((CHUNK, DH), jnp.float32),
        pltpu.VMEM((CHUNK, DH), jnp.float32),
        pltpu.VMEM((CHUNK, DH), jnp.float32),
        pltpu.VMEM((RSLICE, DH), jnp.float32),
        pltpu.VMEM((RSLICE, DH), jnp.float32),
        pltpu.SemaphoreType.DMA,
        pltpu.SemaphoreType.DMA,
        pltpu.SemaphoreType.DMA,
        pltpu.SemaphoreType.DMA,
        pltpu.SemaphoreType.DMA,
        pltpu.SemaphoreType.DMA,
    ),
)
def _sc_jknet(h0, h1, d_hbm, lr_hbm, src_hbm, dst_hbm, z_hbm,
              o_hbm,
              agg_sh, g_sh, d_v, lr_v, srcb, dstb, gbuf0, gbuf1, gbuf2,
              stage_v, fstage_v, gsem0, gsem1, gsem2, ssem0, ssem1, ssem2):
    cid = lax.axis_index("c")
    sid = lax.axis_index("s")
    row0 = sid * ROWS_PER_TILE
    erow0 = sid * CHUNKS_PER_TILE

    pltpu.sync_copy(d_hbm.at[pl.ds(row0, ROWS_PER_TILE)], d_v)
    pltpu.sync_copy(lr_hbm, lr_v)
    pltpu.sync_copy(z_hbm, stage_v)

    def init_g(h_half):
        for k in range(NSLICES):
            rbase = row0 + k * RSLICE
            pltpu.sync_copy(h_half.at[pl.ds(rbase, RSLICE)],
                            gbuf1.at[pl.ds(0, RSLICE)])

            def grow_body(r, _):
                dv = d_v[k * RSLICE + r, :]
                for v in range(DH // 16):
                    cs = pl.ds(v * 16, 16)
                    gbuf0[r, cs] = gbuf1[r, cs] * dv
                return 0

            lax.fori_loop(0, RSLICE, grow_body, 0)
            pltpu.sync_copy(gbuf0.at[pl.ds(0, RSLICE)],
                            g_sh.at[pl.ds(rbase, RSLICE)])
            pltpu.sync_copy(stage_v, agg_sh.at[pl.ds(rbase, RSLICE)])

    pl.when(cid == 0)(lambda: init_g(h0))
    pl.when(cid == 1)(lambda: init_g(h1))
    plsc.subcore_barrier()

    for hop in range(HOPS):
        bufs = (gbuf0, gbuf1, gbuf2)
        gsems = (gsem0, gsem1, gsem2)
        ssems = (ssem0, ssem1, ssem2)
        NB = 3

        def group_body(gi, _):
            pltpu.sync_copy(src_hbm.at[pl.ds(erow0 + gi * GROUP, GROUP)],
                            srcb)
            pltpu.sync_copy(dst_hbm.at[pl.ds(erow0 + gi * GROUP, GROUP)],
                            dstb)
            gp = [pltpu.async_copy(g_sh.at[srcb.at[b]], bufs[b], gsems[b])
                  for b in range(NB)]
            sp = [None] * NB
            for j in range(GROUP):
                b = j % NB
                if j >= 1:
                    bp = (j - 1) % NB
                    sp[bp].wait()
                    if j - 1 + NB < GROUP:
                        gp[bp] = pltpu.async_copy(
                            g_sh.at[srcb.at[j - 1 + NB]], bufs[bp],
                            gsems[bp])
                gp[b].wait()
                sp[b] = pltpu.async_copy(
                    bufs[b], agg_sh.at[dstb.at[j]], ssems[b], add=True)
            sp[(GROUP - 1) % NB].wait()
            return 0

        lax.fori_loop(0, NGROUPS, group_body, 0)
        plsc.subcore_barrier()

        def update(feat_src, cc):
            av = lr_v[hop, :]
            bv = 1.0 - av
            for k in range(NSLICES // 2):
                rbase = row0 + k * 2 * RSLICE
                sl = pl.ds(rbase, 2 * RSLICE)
                pltpu.sync_copy(agg_sh.at[sl], gbuf0)
                pltpu.sync_copy(feat_src.at[sl], gbuf1)

                def row_body(r, _):
                    dv = d_v[k * 2 * RSLICE + r, :]
                    sv = dv * av
                    for v in range(DH // 16):
                        cs = pl.ds(v * 16, 16)
                        nf = gbuf0[r, cs] * sv + gbuf1[r, cs] * bv
                        gbuf1[r, cs] = nf
                        gbuf0[r, cs] = nf * dv
                    return 0

                lax.fori_loop(0, 2 * RSLICE, row_body, 0)
                if hop + 1 < HOPS:
                    pltpu.sync_copy(gbuf0, g_sh.at[sl])
                    pltpu.sync_copy(stage_v, agg_sh.at[pl.ds(rbase, RSLICE)])
                    pltpu.sync_copy(stage_v,
                                    agg_sh.at[pl.ds(rbase + RSLICE, RSLICE)])
                pltpu.sync_copy(gbuf1, o_hbm.at[hop, cc, sl])

        if hop == 0:
            pl.when(cid == 0)(lambda: update(h0, 0))
            pl.when(cid == 1)(lambda: update(h1, 1))
        else:
            pl.when(cid == 0)(lambda: update(o_hbm.at[hop - 1, 0], 0))
            pl.when(cid == 1)(lambda: update(o_hbm.at[hop - 1, 1], 1))
        plsc.subcore_barrier()


def kernel(h, edge_index, d, layer_regular):
    src = edge_index[0]
    dst = edge_index[1]
    pad_e = EPAD - E
    src_p = jnp.concatenate([src, jnp.zeros((pad_e,), jnp.int32)])
    dst_p = jnp.concatenate([dst, jnp.full((pad_e,), N, jnp.int32)])
    srcm = src_p.reshape(NSUB * CHUNKS_PER_TILE, CHUNK)
    dstm = dst_p.reshape(NSUB * CHUNKS_PER_TILE, CHUNK)
    h0 = jnp.pad(h[:, :DH], ((0, NPAD - N), (0, 0)))
    h1 = jnp.pad(h[:, DH:], ((0, NPAD - N), (0, 0)))
    d_pad = jnp.broadcast_to(jnp.pad(d, (0, NPAD - N))[:, None], (NPAD, 16))
    lr_pad = jnp.broadcast_to(layer_regular[:, None], (HOPS, 16))
    zeros = jnp.zeros((RSLICE, DH), jnp.float32)
    o = _sc_jknet(h0, h1, d_pad, lr_pad, srcm, dstm, zeros)
    return o.transpose(2, 0, 1, 3).reshape(NPAD, HOPS * D)[:N]


# speedup vs baseline: 1.0711x; 1.0711x over previous
import functools

import jax
import jax.numpy as jnp
from jax import lax
from jax.experimental import pallas as pl
from jax.experimental.pallas import tpu as pltpu
from jax.experimental.pallas import tpu_sc as plsc

N = 10000
D = 128
DH = 64
HOPS = 4
E = 320000

NSUB = 16
NPAD = 10240
ROWS_PER_TILE = NPAD // NSUB
CHUNK = 128
CHUNKS_PER_TILE = 160
EPAD = NSUB * CHUNKS_PER_TILE * CHUNK
RSLICE = 64
NSLICES = ROWS_PER_TILE // RSLICE
GROUP = 16
NGROUPS = CHUNKS_PER_TILE // GROUP

_mesh = plsc.VectorSubcoreMesh(core_axis_name="c", subcore_axis_name="s")


@functools.partial(
    pl.kernel,
    out_type=jax.ShapeDtypeStruct((NPAD, HOPS * D), jnp.float32),
    mesh=_mesh,
    compiler_params=pltpu.CompilerParams(use_tc_tiling_on_sc=False),
    scratch_types=(
        pltpu.VMEM_SHARED((NPAD, DH), jnp.float32),
        pltpu.VMEM_SHARED((NPAD, DH), jnp.float32),
        pltpu.VMEM((ROWS_PER_TILE, 16), jnp.float32),
        pltpu.VMEM((HOPS, 16), jnp.float32),
        pltpu.VMEM((GROUP, CHUNK), jnp.int32),
        pltpu.VMEM((GROUP, CHUNK), jnp.int32),
        pltpu.VMEM((CHUNK, DH), jnp.float32),
        pltpu.VMEM((CHUNK, DH), jnp.float32),
        pltpu.VMEM((CHUNK, DH), jnp.float32),
        pltpu.VMEM((RSLICE, DH), jnp.float32),
        pltpu.VMEM((RSLICE, DH), jnp.float32),
        pltpu.SemaphoreType.DMA,
        pltpu.SemaphoreType.DMA,
        pltpu.SemaphoreType.DMA,
        pltpu.SemaphoreType.DMA,
        pltpu.SemaphoreType.DMA,
        pltpu.SemaphoreType.DMA,
    ),
)
def _sc_jknet(h0, h1, d_hbm, lr_hbm, src_hbm, dst_hbm, z_hbm,
              o_hbm,
              agg_sh, g_sh, d_v, lr_v, srcb, dstb, gbuf0, gbuf1, gbuf2,
              stage_v, fstage_v, gsem0, gsem1, gsem2, ssem0, ssem1, ssem2):
    cid = lax.axis_index("c")
    sid = lax.axis_index("s")
    row0 = sid * ROWS_PER_TILE
    erow0 = sid * CHUNKS_PER_TILE

    pltpu.sync_copy(d_hbm.at[pl.ds(row0, ROWS_PER_TILE)], d_v)
    pltpu.sync_copy(lr_hbm, lr_v)
    pltpu.sync_copy(z_hbm, stage_v)

    def init_g(h_half):
        for k in range(NSLICES):
            rbase = row0 + k * RSLICE
            pltpu.sync_copy(h_half.at[pl.ds(rbase, RSLICE)],
                            gbuf1.at[pl.ds(0, RSLICE)])

            def grow_body(r, _):
                dv = d_v[k * RSLICE + r, :]
                for v in range(DH // 16):
                    cs = pl.ds(v * 16, 16)
                    gbuf0[r, cs] = gbuf1[r, cs] * dv
                return 0

            lax.fori_loop(0, RSLICE, grow_body, 0)
            pltpu.sync_copy(gbuf0.at[pl.ds(0, RSLICE)],
                            g_sh.at[pl.ds(rbase, RSLICE)])
            pltpu.sync_copy(stage_v, agg_sh.at[pl.ds(rbase, RSLICE)])

    pl.when(cid == 0)(lambda: init_g(h0))
    pl.when(cid == 1)(lambda: init_g(h1))
    plsc.subcore_barrier()

    for hop in range(HOPS):
        bufs = (gbuf0, gbuf1, gbuf2)
        gsems = (gsem0, gsem1, gsem2)
        ssems = (ssem0, ssem1, ssem2)
        NB = 3

        def group_body(gi, _):
            pltpu.sync_copy(src_hbm.at[pl.ds(erow0 + gi * GROUP, GROUP)],
                            srcb)
            pltpu.sync_copy(dst_hbm.at[pl.ds(erow0 + gi * GROUP, GROUP)],
                            dstb)
            gp = [pltpu.async_copy(g_sh.at[srcb.at[b]], bufs[b], gsems[b])
                  for b in range(NB)]
            sp = [None] * NB
            for j in range(GROUP):
                b = j % NB
                if j >= 1:
                    bp = (j - 1) % NB
                    sp[bp].wait()
                    if j - 1 + NB < GROUP:
                        gp[bp] = pltpu.async_copy(
                            g_sh.at[srcb.at[j - 1 + NB]], bufs[bp],
                            gsems[bp])
                gp[b].wait()
                sp[b] = pltpu.async_copy(
                    bufs[b], agg_sh.at[dstb.at[j]], ssems[b], add=True)
            sp[(GROUP - 1) % NB].wait()
            return 0

        lax.fori_loop(0, NGROUPS, group_body, 0)
        plsc.subcore_barrier()

        def update(h_half, cc):
            av = lr_v[hop, :]
            bv = 1.0 - av
            ocol = pl.ds(hop * D + cc * DH, DH)
            fcol = pl.ds((hop - 1) * D + cc * DH, DH)
            for k in range(NSLICES // 2):
                rbase = row0 + k * 2 * RSLICE
                sl = pl.ds(rbase, 2 * RSLICE)
                pltpu.sync_copy(agg_sh.at[sl], gbuf0)
                if hop == 0:
                    pltpu.sync_copy(h_half.at[sl], gbuf1)
                else:
                    pltpu.sync_copy(o_hbm.at[sl, fcol], gbuf1)

                def row_body(r2, _):
                    for u in range(2):
                        r = 2 * r2 + u
                        dv = d_v[k * 2 * RSLICE + r, :]
                        sv = dv * av
                        for v in range(DH // 16):
                            cs = pl.ds(v * 16, 16)
                            nf = gbuf0[r, cs] * sv + gbuf1[r, cs] * bv
                            gbuf1[r, cs] = nf
                            gbuf0[r, cs] = nf * dv
                    return 0

                lax.fori_loop(0, RSLICE, row_body, 0)
                if hop + 1 < HOPS:
                    pltpu.sync_copy(gbuf0, g_sh.at[sl])
                    pltpu.sync_copy(stage_v, agg_sh.at[pl.ds(rbase, RSLICE)])
                    pltpu.sync_copy(stage_v,
                                    agg_sh.at[pl.ds(rbase + RSLICE, RSLICE)])
                pltpu.sync_copy(gbuf1, o_hbm.at[sl, ocol])

        pl.when(cid == 0)(lambda: update(h0, 0))
        pl.when(cid == 1)(lambda: update(h1, 1))
        plsc.subcore_barrier()


def kernel(h, edge_index, d, layer_regular):
    src = edge_index[0]
    dst = edge_index[1]
    pad_e = EPAD - E
    src_p = jnp.concatenate([src, jnp.zeros((pad_e,), jnp.int32)])
    dst_p = jnp.concatenate([dst, jnp.full((pad_e,), N, jnp.int32)])
    srcm = src_p.reshape(NSUB * CHUNKS_PER_TILE, CHUNK)
    dstm = dst_p.reshape(NSUB * CHUNKS_PER_TILE, CHUNK)
    h0 = jnp.pad(h[:, :DH], ((0, NPAD - N), (0, 0)))
    h1 = jnp.pad(h[:, DH:], ((0, NPAD - N), (0, 0)))
    d_pad = jnp.broadcast_to(jnp.pad(d, (0, NPAD - N))[:, None], (NPAD, 16))
    lr_pad = jnp.broadcast_to(layer_regular[:, None], (HOPS, 16))
    zeros = jnp.zeros((RSLICE, DH), jnp.float32)
    o = _sc_jknet(h0, h1, d_pad, lr_pad, srcm, dstm, zeros)
    return o[:N]


# speedup vs baseline: 1.1466x; 1.0705x over previous
import functools

import jax
import jax.numpy as jnp
from jax import lax
from jax.experimental import pallas as pl
from jax.experimental.pallas import tpu as pltpu
from jax.experimental.pallas import tpu_sc as plsc

N = 10000
D = 128
DH = 64
HOPS = 4
E = 320000

NSUB = 16
NPAD = 10240
ROWS_PER_TILE = NPAD // NSUB
CHUNK = 128
CHUNKS_PER_TILE = 160
EPAD = NSUB * CHUNKS_PER_TILE * CHUNK
RSLICE = 64
NSLICES = ROWS_PER_TILE // RSLICE
GROUP = 32
NGROUPS = CHUNKS_PER_TILE // GROUP

_mesh = plsc.VectorSubcoreMesh(core_axis_name="c", subcore_axis_name="s")


@functools.partial(
    pl.kernel,
    out_type=jax.ShapeDtypeStruct((NPAD, HOPS * D), jnp.float32),
    mesh=_mesh,
    compiler_params=pltpu.CompilerParams(use_tc_tiling_on_sc=False),
    scratch_types=(
        pltpu.VMEM_SHARED((NPAD, DH), jnp.float32),
        pltpu.VMEM_SHARED((NPAD, DH), jnp.float32),
        pltpu.VMEM((ROWS_PER_TILE, 16), jnp.float32),
        pltpu.VMEM((HOPS, 16), jnp.float32),
        pltpu.VMEM((GROUP, CHUNK), jnp.int32),
        pltpu.VMEM((GROUP, CHUNK), jnp.int32),
        pltpu.VMEM((CHUNK, DH), jnp.float32),
        pltpu.VMEM((CHUNK, DH), jnp.float32),
        pltpu.VMEM((CHUNK, DH), jnp.float32),
        pltpu.VMEM((RSLICE, DH), jnp.float32),
        pltpu.SemaphoreType.DMA,
        pltpu.SemaphoreType.DMA,
        pltpu.SemaphoreType.DMA,
        pltpu.SemaphoreType.DMA,
        pltpu.SemaphoreType.DMA,
        pltpu.SemaphoreType.DMA,
    ),
)
def _sc_jknet(h0, h1, d_hbm, lr_hbm, src_hbm, dst_hbm, z_hbm,
              o_hbm,
              agg_sh, g_sh, d_v, lr_v, srcb, dstb, gbuf0, gbuf1, gbuf2,
              stage_v, gsem0, gsem1, gsem2, ssem0, ssem1, ssem2):
    cid = lax.axis_index("c")
    sid = lax.axis_index("s")
    row0 = sid * ROWS_PER_TILE
    erow0 = sid * CHUNKS_PER_TILE

    pltpu.sync_copy(d_hbm.at[pl.ds(row0, ROWS_PER_TILE)], d_v)
    pltpu.sync_copy(lr_hbm, lr_v)
    pltpu.sync_copy(z_hbm, stage_v)

    def init_g(h_half):
        for k in range(NSLICES):
            rbase = row0 + k * RSLICE
            pltpu.sync_copy(h_half.at[pl.ds(rbase, RSLICE)],
                            gbuf1.at[pl.ds(0, RSLICE)])

            def grow_body(r, _):
                dv = d_v[k * RSLICE + r, :]
                for v in range(DH // 16):
                    cs = pl.ds(v * 16, 16)
                    gbuf0[r, cs] = gbuf1[r, cs] * dv
                return 0

            lax.fori_loop(0, RSLICE, grow_body, 0)
            pltpu.sync_copy(gbuf0.at[pl.ds(0, RSLICE)],
                            g_sh.at[pl.ds(rbase, RSLICE)])
            pltpu.sync_copy(stage_v, agg_sh.at[pl.ds(rbase, RSLICE)])

    pl.when(cid == 0)(lambda: init_g(h0))
    pl.when(cid == 1)(lambda: init_g(h1))
    plsc.subcore_barrier()

    for hop in range(HOPS):
        bufs = (gbuf0, gbuf1, gbuf2)
        gsems = (gsem0, gsem1, gsem2)
        ssems = (ssem0, ssem1, ssem2)
        NB = 3

        def group_body(gi, _):
            pltpu.sync_copy(src_hbm.at[pl.ds(erow0 + gi * GROUP, GROUP)],
                            srcb)
            pltpu.sync_copy(dst_hbm.at[pl.ds(erow0 + gi * GROUP, GROUP)],
                            dstb)
            gp = [pltpu.async_copy(g_sh.at[srcb.at[b]], bufs[b], gsems[b])
                  for b in range(NB)]
            sp = [None] * NB
            for j in range(GROUP):
                b = j % NB
                if j >= 1:
                    bp = (j - 1) % NB
                    sp[bp].wait()
                    if j - 1 + NB < GROUP:
                        gp[bp] = pltpu.async_copy(
                            g_sh.at[srcb.at[j - 1 + NB]], bufs[bp],
                            gsems[bp])
                gp[b].wait()
                sp[b] = pltpu.async_copy(
                    bufs[b], agg_sh.at[dstb.at[j]], ssems[b], add=True)
            sp[(GROUP - 1) % NB].wait()
            return 0

        lax.fori_loop(0, NGROUPS, group_body, 0)
        plsc.subcore_barrier()

        def update(h_half, cc):
            av = lr_v[hop, :]
            bv = 1.0 - av
            ocol = pl.ds(hop * D + cc * DH, DH)
            fcol = pl.ds((hop - 1) * D + cc * DH, DH)
            for k in range(NSLICES // 2):
                rbase = row0 + k * 2 * RSLICE
                sl = pl.ds(rbase, 2 * RSLICE)
                pltpu.sync_copy(agg_sh.at[sl], gbuf0)
                if hop == 0:
                    pltpu.sync_copy(h_half.at[sl], gbuf1)
                else:
                    pltpu.sync_copy(o_hbm.at[sl, fcol], gbuf1)

                def row_body(r2, _):
                    for u in range(2):
                        r = 2 * r2 + u
                        dv = d_v[k * 2 * RSLICE + r, :]
                        sv = dv * av
                        for v in range(DH // 16):
                            cs = pl.ds(v * 16, 16)
                            nf = gbuf0[r, cs] * sv + gbuf1[r, cs] * bv
                            gbuf1[r, cs] = nf
                            gbuf0[r, cs] = nf * dv
                    return 0

                lax.fori_loop(0, RSLICE, row_body, 0)
                if hop + 1 < HOPS:
                    pltpu.sync_copy(gbuf0, g_sh.at[sl])
                    pltpu.sync_copy(stage_v, agg_sh.at[pl.ds(rbase, RSLICE)])
                    pltpu.sync_copy(stage_v,
                                    agg_sh.at[pl.ds(rbase + RSLICE, RSLICE)])
                pltpu.sync_copy(gbuf1, o_hbm.at[sl, ocol])

        pl.when(cid == 0)(lambda: update(h0, 0))
        pl.when(cid == 1)(lambda: update(h1, 1))
        plsc.subcore_barrier()


def kernel(h, edge_index, d, layer_regular):
    src = edge_index[0]
    dst = edge_index[1]
    pad_e = EPAD - E
    src_p = jnp.concatenate([src, jnp.zeros((pad_e,), jnp.int32)])
    dst_p = jnp.concatenate([dst, jnp.full((pad_e,), N, jnp.int32)])
    srcm = src_p.reshape(NSUB * CHUNKS_PER_TILE, CHUNK)
    dstm = dst_p.reshape(NSUB * CHUNKS_PER_TILE, CHUNK)
    h0 = jnp.pad(h[:, :DH], ((0, NPAD - N), (0, 0)))
    h1 = jnp.pad(h[:, DH:], ((0, NPAD - N), (0, 0)))
    d_pad = jnp.broadcast_to(jnp.pad(d, (0, NPAD - N))[:, None], (NPAD, 16))
    lr_pad = jnp.broadcast_to(layer_regular[:, None], (HOPS, 16))
    zeros = jnp.zeros((RSLICE, DH), jnp.float32)
    o = _sc_jknet(h0, h1, d_pad, lr_pad, srcm, dstm, zeros)
    return o[:N]


# speedup vs baseline: 1.2030x; 1.0492x over previous
import functools

import jax
import jax.numpy as jnp
from jax import lax
from jax.experimental import pallas as pl
from jax.experimental.pallas import tpu as pltpu
from jax.experimental.pallas import tpu_sc as plsc

N = 10000
D = 128
DH = 64
HOPS = 4
E = 320000

NSUB = 16
NPAD = 10240
ROWS_PER_TILE = NPAD // NSUB
CHUNK = 128
CHUNKS_PER_TILE = 160
EPAD = NSUB * CHUNKS_PER_TILE * CHUNK
RSLICE = 64
UROWS = N // NSUB
USLICE = 125
UNS = UROWS // USLICE
GROUP = 32
NGROUPS = CHUNKS_PER_TILE // GROUP

_mesh = plsc.VectorSubcoreMesh(core_axis_name="c", subcore_axis_name="s")


@functools.partial(
    pl.kernel,
    out_type=jax.ShapeDtypeStruct((N, HOPS * D), jnp.float32),
    mesh=_mesh,
    compiler_params=pltpu.CompilerParams(use_tc_tiling_on_sc=False),
    scratch_types=(
        pltpu.VMEM_SHARED((NPAD, DH), jnp.float32),
        pltpu.VMEM_SHARED((NPAD, DH), jnp.float32),
        pltpu.VMEM((UROWS, 16), jnp.float32),
        pltpu.VMEM((HOPS, 16), jnp.float32),
        pltpu.VMEM((GROUP, CHUNK), jnp.int32),
        pltpu.VMEM((GROUP, CHUNK), jnp.int32),
        pltpu.VMEM((CHUNK, DH), jnp.float32),
        pltpu.VMEM((CHUNK, DH), jnp.float32),
        pltpu.VMEM((CHUNK, DH), jnp.float32),
        pltpu.VMEM((RSLICE, DH), jnp.float32),
        pltpu.SemaphoreType.DMA,
        pltpu.SemaphoreType.DMA,
        pltpu.SemaphoreType.DMA,
        pltpu.SemaphoreType.DMA,
        pltpu.SemaphoreType.DMA,
        pltpu.SemaphoreType.DMA,
    ),
)
def _sc_jknet(h_hbm, d_hbm, lr_hbm, src_hbm, dst_hbm, z_hbm,
              o_hbm,
              agg_sh, g_sh, d_v, lr_v, srcb, dstb, gbuf0, gbuf1, gbuf2,
              stage_v, gsem0, gsem1, gsem2, ssem0, ssem1, ssem2):
    cid = lax.axis_index("c")
    sid = lax.axis_index("s")
    row0 = sid * UROWS
    erow0 = sid * CHUNKS_PER_TILE

    pltpu.sync_copy(d_hbm.at[pl.ds(row0, UROWS)], d_v)
    pltpu.sync_copy(lr_hbm, lr_v)
    pltpu.sync_copy(z_hbm, stage_v)

    def zero_agg(rbase):
        pltpu.sync_copy(stage_v, agg_sh.at[pl.ds(rbase, RSLICE)])
        pltpu.sync_copy(stage_v.at[pl.ds(0, USLICE - RSLICE)],
                        agg_sh.at[pl.ds(rbase + RSLICE, USLICE - RSLICE)])

    def init_g(cc):
        hcol = pl.ds(cc * DH, DH)

        def slice_body(k, _):
            rbase = row0 + k * USLICE
            pltpu.sync_copy(h_hbm.at[pl.ds(rbase, USLICE), hcol],
                            gbuf1.at[pl.ds(0, USLICE)])

            def grow_body(r5, _):
                for u in range(5):
                    r = 5 * r5 + u
                    dv = d_v[k * USLICE + r, :]
                    for v in range(DH // 16):
                        cs = pl.ds(v * 16, 16)
                        gbuf0[r, cs] = gbuf1[r, cs] * dv
                return 0

            lax.fori_loop(0, USLICE // 5, grow_body, 0)
            pltpu.sync_copy(gbuf0.at[pl.ds(0, USLICE)],
                            g_sh.at[pl.ds(rbase, USLICE)])
            zero_agg(rbase)
            return 0

        lax.fori_loop(0, UNS, slice_body, 0)

    pl.when(cid == 0)(lambda: init_g(0))
    pl.when(cid == 1)(lambda: init_g(1))
    plsc.subcore_barrier()

    for hop in range(HOPS):
        bufs = (gbuf0, gbuf1, gbuf2)
        gsems = (gsem0, gsem1, gsem2)
        ssems = (ssem0, ssem1, ssem2)
        NB = 3

        def group_body(gi, _):
            pltpu.sync_copy(src_hbm.at[pl.ds(erow0 + gi * GROUP, GROUP)],
                            srcb)
            pltpu.sync_copy(dst_hbm.at[pl.ds(erow0 + gi * GROUP, GROUP)],
                            dstb)
            gp = [pltpu.async_copy(g_sh.at[srcb.at[b]], bufs[b], gsems[b])
                  for b in range(NB)]
            sp = [None] * NB
            for j in range(GROUP):
                b = j % NB
                if j >= 1:
                    bp = (j - 1) % NB
                    sp[bp].wait()
                    if j - 1 + NB < GROUP:
                        gp[bp] = pltpu.async_copy(
                            g_sh.at[srcb.at[j - 1 + NB]], bufs[bp],
                            gsems[bp])
                gp[b].wait()
                sp[b] = pltpu.async_copy(
                    bufs[b], agg_sh.at[dstb.at[j]], ssems[b], add=True)
            sp[(GROUP - 1) % NB].wait()
            return 0

        lax.fori_loop(0, NGROUPS, group_body, 0)
        plsc.subcore_barrier()

        def update(cc):
            av = lr_v[hop, :]
            bv = 1.0 - av
            hcol = pl.ds(cc * DH, DH)
            ocol = pl.ds(hop * D + cc * DH, DH)
            fcol = pl.ds((hop - 1) * D + cc * DH, DH)
            def slice_body(k, _):
                rbase = row0 + k * USLICE
                sl = pl.ds(rbase, USLICE)
                part = pl.ds(0, USLICE)
                pltpu.sync_copy(agg_sh.at[sl], gbuf0.at[part])
                if hop == 0:
                    pltpu.sync_copy(h_hbm.at[sl, hcol], gbuf1.at[part])
                else:
                    pltpu.sync_copy(o_hbm.at[sl, fcol], gbuf1.at[part])

                def row_body(r5, _):
                    for u in range(5):
                        r = 5 * r5 + u
                        dv = d_v[k * USLICE + r, :]
                        sv = dv * av
                        for v in range(DH // 16):
                            cs = pl.ds(v * 16, 16)
                            nf = gbuf0[r, cs] * sv + gbuf1[r, cs] * bv
                            gbuf1[r, cs] = nf
                            gbuf0[r, cs] = nf * dv
                    return 0

                lax.fori_loop(0, USLICE // 5, row_body, 0)
                if hop + 1 < HOPS:
                    pltpu.sync_copy(gbuf0.at[part], g_sh.at[sl])
                    zero_agg(rbase)
                pltpu.sync_copy(gbuf1.at[part], o_hbm.at[sl, ocol])
                return 0

            lax.fori_loop(0, UNS, slice_body, 0)

        pl.when(cid == 0)(lambda: update(0))
        pl.when(cid == 1)(lambda: update(1))
        plsc.subcore_barrier()


def kernel(h, edge_index, d, layer_regular):
    src = edge_index[0]
    dst = edge_index[1]
    pad_e = EPAD - E
    src_p = jnp.concatenate([src, jnp.zeros((pad_e,), jnp.int32)])
    dst_p = jnp.concatenate([dst, jnp.full((pad_e,), N, jnp.int32)])
    srcm = src_p.reshape(NSUB * CHUNKS_PER_TILE, CHUNK)
    dstm = dst_p.reshape(NSUB * CHUNKS_PER_TILE, CHUNK)
    d_bc = jnp.broadcast_to(d[:, None], (N, 16))
    lr_pad = jnp.broadcast_to(layer_regular[:, None], (HOPS, 16))
    zeros = jnp.zeros((RSLICE, DH), jnp.float32)
    return _sc_jknet(h, d_bc, lr_pad, srcm, dstm, zeros)


# speedup vs baseline: 1.2220x; 1.0158x over previous
import functools

import jax
import jax.numpy as jnp
from jax import lax
from jax.experimental import pallas as pl
from jax.experimental.pallas import tpu as pltpu
from jax.experimental.pallas import tpu_sc as plsc

N = 10000
D = 128
DH = 64
HOPS = 4
E = 320000

NSUB = 16
CHUNK = 125
CHUNKS_PER_TILE = 160
RSLICE = 64
UROWS = N // NSUB
USLICE = 125
UNS = UROWS // USLICE
GROUP = 32
NGROUPS = CHUNKS_PER_TILE // GROUP

_mesh = plsc.VectorSubcoreMesh(core_axis_name="c", subcore_axis_name="s")


@functools.partial(
    pl.kernel,
    out_type=jax.ShapeDtypeStruct((N, HOPS * D), jnp.float32),
    mesh=_mesh,
    compiler_params=pltpu.CompilerParams(use_tc_tiling_on_sc=False),
    scratch_types=(
        pltpu.VMEM_SHARED((N, DH), jnp.float32),
        pltpu.VMEM_SHARED((N, DH), jnp.float32),
        pltpu.VMEM((UROWS, 16), jnp.float32),
        pltpu.VMEM((HOPS, 16), jnp.float32),
        pltpu.VMEM((GROUP, CHUNK), jnp.int32),
        pltpu.VMEM((GROUP, CHUNK), jnp.int32),
        pltpu.VMEM((CHUNK, DH), jnp.float32),
        pltpu.VMEM((CHUNK, DH), jnp.float32),
        pltpu.VMEM((CHUNK, DH), jnp.float32),
        pltpu.VMEM((RSLICE, DH), jnp.float32),
        pltpu.SemaphoreType.DMA,
        pltpu.SemaphoreType.DMA,
        pltpu.SemaphoreType.DMA,
        pltpu.SemaphoreType.DMA,
        pltpu.SemaphoreType.DMA,
        pltpu.SemaphoreType.DMA,
    ),
)
def _sc_jknet(h_hbm, d_hbm, lr_hbm, src_hbm, dst_hbm, z_hbm,
              o_hbm,
              agg_sh, g_sh, d_v, lr_v, srcb, dstb, gbuf0, gbuf1, gbuf2,
              stage_v, gsem0, gsem1, gsem2, ssem0, ssem1, ssem2):
    cid = lax.axis_index("c")
    sid = lax.axis_index("s")
    row0 = sid * UROWS
    erow0 = sid * CHUNKS_PER_TILE

    pltpu.sync_copy(d_hbm.at[pl.ds(row0, UROWS)], d_v)
    pltpu.sync_copy(lr_hbm, lr_v)
    pltpu.sync_copy(z_hbm, stage_v)

    def zero_agg(rbase):
        pltpu.sync_copy(stage_v, agg_sh.at[pl.ds(rbase, RSLICE)])
        pltpu.sync_copy(stage_v.at[pl.ds(0, USLICE - RSLICE)],
                        agg_sh.at[pl.ds(rbase + RSLICE, USLICE - RSLICE)])

    def init_g(cc):
        hcol = pl.ds(cc * DH, DH)

        def slice_body(k, _):
            rbase = row0 + k * USLICE
            pltpu.sync_copy(h_hbm.at[pl.ds(rbase, USLICE), hcol],
                            gbuf1.at[pl.ds(0, USLICE)])

            def grow_body(r5, _):
                for u in range(5):
                    r = 5 * r5 + u
                    dv = d_v[k * USLICE + r, :]
                    for v in range(DH // 16):
                        cs = pl.ds(v * 16, 16)
                        gbuf0[r, cs] = gbuf1[r, cs] * dv
                return 0

            lax.fori_loop(0, USLICE // 5, grow_body, 0)
            pltpu.sync_copy(gbuf0.at[pl.ds(0, USLICE)],
                            g_sh.at[pl.ds(rbase, USLICE)])
            zero_agg(rbase)
            return 0

        lax.fori_loop(0, UNS, slice_body, 0)

    pl.when(cid == 0)(lambda: init_g(0))
    pl.when(cid == 1)(lambda: init_g(1))
    plsc.subcore_barrier()

    for hop in range(HOPS):
        bufs = (gbuf0, gbuf1, gbuf2)
        gsems = (gsem0, gsem1, gsem2)
        ssems = (ssem0, ssem1, ssem2)
        NB = 3

        def group_body(gi, _):
            pltpu.sync_copy(src_hbm.at[pl.ds(erow0 + gi * GROUP, GROUP)],
                            srcb)
            pltpu.sync_copy(dst_hbm.at[pl.ds(erow0 + gi * GROUP, GROUP)],
                            dstb)
            cpart = pl.ds(0, CHUNK)
            gp = [pltpu.async_copy(g_sh.at[srcb.at[b]], bufs[b].at[cpart],
                                   gsems[b])
                  for b in range(NB)]
            sp = [None] * NB
            for j in range(GROUP):
                b = j % NB
                if j >= 1:
                    bp = (j - 1) % NB
                    sp[bp].wait()
                    if j - 1 + NB < GROUP:
                        gp[bp] = pltpu.async_copy(
                            g_sh.at[srcb.at[j - 1 + NB]], bufs[bp].at[cpart],
                            gsems[bp])
                gp[b].wait()
                sp[b] = pltpu.async_copy(
                    bufs[b].at[cpart], agg_sh.at[dstb.at[j]], ssems[b],
                    add=True)
            sp[(GROUP - 1) % NB].wait()
            return 0

        lax.fori_loop(0, NGROUPS, group_body, 0)
        plsc.subcore_barrier()

        def update(cc):
            av = lr_v[hop, :]
            bv = 1.0 - av
            hcol = pl.ds(cc * DH, DH)
            ocol = pl.ds(hop * D + cc * DH, DH)
            fcol = pl.ds((hop - 1) * D + cc * DH, DH)
            def slice_body(k, _):
                rbase = row0 + k * USLICE
                sl = pl.ds(rbase, USLICE)
                part = pl.ds(0, USLICE)
                pltpu.sync_copy(agg_sh.at[sl], gbuf0.at[part])
                if hop == 0:
                    pltpu.sync_copy(h_hbm.at[sl, hcol], gbuf1.at[part])
                else:
                    pltpu.sync_copy(o_hbm.at[sl, fcol], gbuf1.at[part])

                def row_body(r5, _):
                    for u in range(5):
                        r = 5 * r5 + u
                        dv = d_v[k * USLICE + r, :]
                        sv = dv * av
                        for v in range(DH // 16):
                            cs = pl.ds(v * 16, 16)
                            nf = gbuf0[r, cs] * sv + gbuf1[r, cs] * bv
                            gbuf1[r, cs] = nf
                            gbuf0[r, cs] = nf * dv
                    return 0

                lax.fori_loop(0, USLICE // 5, row_body, 0)
                if hop + 1 < HOPS:
                    pltpu.sync_copy(gbuf0.at[part], g_sh.at[sl])
                    zero_agg(rbase)
                pltpu.sync_copy(gbuf1.at[part], o_hbm.at[sl, ocol])
                return 0

            lax.fori_loop(0, UNS, slice_body, 0)

        pl.when(cid == 0)(lambda: update(0))
        pl.when(cid == 1)(lambda: update(1))
        plsc.subcore_barrier()


def kernel(h, edge_index, d, layer_regular):
    srcm = edge_index[0].reshape(NSUB * CHUNKS_PER_TILE, CHUNK)
    dstm = edge_index[1].reshape(NSUB * CHUNKS_PER_TILE, CHUNK)
    d_bc = jnp.broadcast_to(d[:, None], (N, 16))
    lr_pad = jnp.broadcast_to(layer_regular[:, None], (HOPS, 16))
    zeros = jnp.zeros((RSLICE, DH), jnp.float32)
    return _sc_jknet(h, d_bc, lr_pad, srcm, dstm, zeros)
